# baseline (speedup 1.0000x reference)
import functools

import jax
import jax.numpy as jnp
import numpy as np
from jax import lax
from jax.experimental import pallas as pl
from jax.experimental.pallas import tpu as pltpu
from jax.experimental.pallas import tpu_sc as plsc

N_NODES = 10000
N_EDGES = 10000
NNZ = 320000
D = 128
EPS = 1e-5

_NC, _NS = 2, 16
_W = 128
_NWIN = NNZ // _W
_NWK = _NC * _NS
_WPW = _NWIN // _NWK
_XTRA = _NWIN - _WPW * _NWK
_ACC_ROWS = N_EDGES
_ZCH = _ACC_ROWS // _NS
_CCH = 640
_NBUF = 3

_HIGH = jax.lax.Precision.HIGHEST


def _spmm_partials(msg, idx_win, zrows):
    mesh = plsc.VectorSubcoreMesh(core_axis_name="c", subcore_axis_name="s")

    def body(msg_hbm, idx_hbm, z_hbm, out_hbm, ic, r0, r1, r2,
             g0, g1, g2, s0, s1, s2, acc):
        rows = [r0, r1, r2]
        gsems = [g0, g1, g2]
        ssems = [s0, s1, s2]
        c = lax.axis_index("c")
        s = lax.axis_index("s")
        wid = s * _NC + c
        base = wid * _WPW
        pltpu.sync_copy(z_hbm, acc.at[pl.ds(s * _ZCH, _ZCH)])
        plsc.subcore_barrier()

        def _scatter_wait(b):
            pltpu.make_async_copy(rows[b], acc.at[ic.at[b, 1]],
                                  ssems[b]).wait()

        @pl.loop(0, _WPW, step=_NBUF)
        def _(i):
            @pl.when(i > 0)
            def _():
                for b in range(_NBUF):
                    _scatter_wait(b)

            pltpu.sync_copy(idx_hbm.at[pl.ds(base + i, _NBUF)], ic)
            g = [pltpu.async_copy(msg_hbm.at[ic.at[b, 0]], rows[b], gsems[b])
                 for b in range(_NBUF)]
            for b in range(_NBUF):
                g[b].wait()
                pltpu.async_copy(rows[b], acc.at[ic.at[b, 1]],
                                 ssems[b], add=True)

        for b in range(_NBUF):
            _scatter_wait(b)

        @pl.when(wid < _XTRA)
        def _():
            j = _NWK * _WPW + wid
            pltpu.sync_copy(idx_hbm.at[pl.ds(j, 1)], ic.at[pl.ds(0, 1)])
            pltpu.sync_copy(msg_hbm.at[ic.at[0, 0]], r0)
            pltpu.sync_copy(r0, acc.at[ic.at[0, 1]], add=True)

        plsc.subcore_barrier()

        @pl.when(s < _NS - 1)
        def _():
            pltpu.sync_copy(acc.at[pl.ds(s * _CCH, _CCH)],
                            out_hbm.at[c].at[pl.ds(s * _CCH, _CCH)])

        @pl.when(s == _NS - 1)
        def _():
            last = (_NS - 1) * _CCH
            pltpu.sync_copy(acc.at[pl.ds(last, N_EDGES - last)],
                            out_hbm.at[c].at[pl.ds(last, N_EDGES - last)])

    k = pl.kernel(
        body,
        out_type=jax.ShapeDtypeStruct((_NC, N_EDGES, D), jnp.float32),
        mesh=mesh,
        scratch_types=[
            pltpu.VMEM((_NBUF, 2, _W), jnp.int32),
            pltpu.VMEM((_W, D), jnp.float32),
            pltpu.VMEM((_W, D), jnp.float32),
            pltpu.VMEM((_W, D), jnp.float32),
            pltpu.SemaphoreType.DMA,
            pltpu.SemaphoreType.DMA,
            pltpu.SemaphoreType.DMA,
            pltpu.SemaphoreType.DMA,
            pltpu.SemaphoreType.DMA,
            pltpu.SemaphoreType.DMA,
            pltpu.VMEM_SHARED((_ACC_ROWS, D), jnp.float32),
        ],
    )
    return k(msg, idx_win, zrows)


_RB = 1000
_GRID = N_NODES // _RB


def _pre_body(x0_ref, x1_ref, wh_ref, bh_ref, wn_ref, bn_ref,
              h0_ref, h1_ref, msg_ref):
    h0 = jnp.dot(x0_ref[...], wh_ref[...], precision=_HIGH) + bh_ref[...]
    h1 = jnp.dot(x1_ref[...], wh_ref[...], precision=_HIGH) + bh_ref[...]
    h0_ref[...] = h0
    h1_ref[...] = h1
    msg_ref[...] = jax.nn.sigmoid(
        jnp.dot(h0, wn_ref[...], precision=_HIGH) + bn_ref[...])


def _pre(x_0, x_1, W_hid, b_hid, Wn, bn):
    row = lambda i: (i, 0)
    fixed = lambda i: (0, 0)
    return pl.pallas_call(
        _pre_body,
        grid=(_GRID,),
        in_specs=[
            pl.BlockSpec((_RB, D), row), pl.BlockSpec((_RB, D), row),
            pl.BlockSpec((D, D), fixed), pl.BlockSpec((1, D), fixed),
            pl.BlockSpec((D, D), fixed), pl.BlockSpec((1, D), fixed),
        ],
        out_specs=[pl.BlockSpec((_RB, D), row)] * 3,
        out_shape=[jax.ShapeDtypeStruct((N_NODES, D), jnp.float32)] * 3,
    )(x_0, x_1, W_hid, b_hid, Wn, bn)


def _he_body(x1_ref, p_ref, wa_ref, wb_ref, bhe_ref, msg_ref, m1_ref):
    m1 = p_ref[0] + p_ref[1]
    m1_ref[...] = m1
    msg_ref[...] = jax.nn.sigmoid(
        jnp.dot(x1_ref[...], wa_ref[...], precision=_HIGH)
        + jnp.dot(m1, wb_ref[...], precision=_HIGH) + bhe_ref[...])


def _he(x1, p, Whe_a, Whe_b, bhe):
    row = lambda i: (i, 0)
    fixed = lambda i: (0, 0)
    return pl.pallas_call(
        _he_body,
        grid=(_GRID,),
        in_specs=[
            pl.BlockSpec((_RB, D), row),
            pl.BlockSpec((_NC, _RB, D), lambda i: (0, i, 0)),
            pl.BlockSpec((D, D), fixed), pl.BlockSpec((D, D), fixed),
            pl.BlockSpec((1, D), fixed),
        ],
        out_specs=[pl.BlockSpec((_RB, D), row)] * 2,
        out_shape=[jax.ShapeDtypeStruct((N_EDGES, D), jnp.float32)] * 2,
    )(x1, p, Whe_a, Whe_b, bhe)


_BN_SCALE = float(1.0 / np.sqrt(1.0 + EPS))


def _upd_body(x0_ref, p_ref, gn_ref, gnb_ref, x1_ref, m1_ref, gh_ref, ghb_ref,
              wn_ref, bn_ref, x0o_ref, x1o_ref, msg_ref):
    x0n = jax.nn.sigmoid(x0_ref[...] * (gn_ref[...] * _BN_SCALE) + gnb_ref[...]
                         + p_ref[0] + p_ref[1])
    x1n = jax.nn.sigmoid(x1_ref[...] * (gh_ref[...] * _BN_SCALE) + ghb_ref[...]
                         + m1_ref[...])
    x0o_ref[...] = x0n
    x1o_ref[...] = x1n
    msg_ref[...] = jax.nn.sigmoid(
        jnp.dot(x0n, wn_ref[...], precision=_HIGH) + bn_ref[...])


def _upd(x0, p, gn, gnb, x1, m1, gh, ghb, Wn, bn):
    row = lambda i: (i, 0)
    fixed = lambda i: (0, 0)
    return pl.pallas_call(
        _upd_body,
        grid=(_GRID,),
        in_specs=[
            pl.BlockSpec((_RB, D), row),
            pl.BlockSpec((_NC, _RB, D), lambda i: (0, i, 0)),
            pl.BlockSpec((1, D), fixed), pl.BlockSpec((1, D), fixed),
            pl.BlockSpec((_RB, D), row), pl.BlockSpec((_RB, D), row),
            pl.BlockSpec((1, D), fixed), pl.BlockSpec((1, D), fixed),
            pl.BlockSpec((D, D), fixed), pl.BlockSpec((1, D), fixed),
        ],
        out_specs=[pl.BlockSpec((_RB, D), row)] * 3,
        out_shape=[jax.ShapeDtypeStruct((N_NODES, D), jnp.float32)] * 3,
    )(x0, p, gn, gnb, x1, m1, gh, ghb, Wn, bn)


def _fin_body(x0_ref, p_ref, gn_ref, gnb_ref, wo_ref, bo_ref, out_ref):
    x0n = jax.nn.sigmoid(x0_ref[...] * (gn_ref[...] * _BN_SCALE) + gnb_ref[...]
                         + p_ref[0] + p_ref[1])
    out_ref[...] = jnp.dot(x0n, wo_ref[...], precision=_HIGH) + bo_ref[...]


def _fin(x0, p, gn, gnb, Wo_pad, bo_pad):
    row = lambda i: (i, 0)
    fixed = lambda i: (0, 0)
    return pl.pallas_call(
        _fin_body,
        grid=(_GRID,),
        in_specs=[
            pl.BlockSpec((_RB, D), row),
            pl.BlockSpec((_NC, _RB, D), lambda i: (0, i, 0)),
            pl.BlockSpec((1, D), fixed), pl.BlockSpec((1, D), fixed),
            pl.BlockSpec((D, D), fixed), pl.BlockSpec((1, D), fixed),
        ],
        out_specs=pl.BlockSpec((_RB, D), row),
        out_shape=jax.ShapeDtypeStruct((N_NODES, D), jnp.float32),
    )(x0, p, gn, gnb, Wo_pad, bo_pad)


def kernel(x_0, x_1, W_hid, b_hid, Wn0, bn0, Whe0, bhe0, gamma_n0, beta_n0,
           gamma_h0, beta_h0, Wn1, bn1, Whe1, bhe1, gamma_n1, beta_n1,
           gamma_h1, beta_h1, W_out, b_out, incidence_1):
    node_idx = incidence_1[0]
    he_idx = incidence_1[1]

    def _windows(src, dst):
        return jnp.stack([src.reshape(_NWIN, _W), dst.reshape(_NWIN, _W)],
                         axis=1)

    idxA = _windows(node_idx, he_idx)
    idxB = _windows(he_idx, node_idx)
    zrows = jnp.zeros((_ZCH, D), jnp.float32)

    b_hid_r = b_hid.reshape(1, D)
    bn0_r = bn0.reshape(1, D)
    bn1_r = bn1.reshape(1, D)
    bhe0_r = bhe0.reshape(1, D)
    bhe1_r = bhe1.reshape(1, D)
    gn0_r = gamma_n0.reshape(1, D)
    gnb0_r = beta_n0.reshape(1, D)
    gh0_r = gamma_h0.reshape(1, D)
    ghb0_r = beta_h0.reshape(1, D)
    gn1_r = gamma_n1.reshape(1, D)
    gnb1_r = beta_n1.reshape(1, D)
    gh1_r = gamma_h1.reshape(1, D)
    ghb1_r = beta_h1.reshape(1, D)
    Whe0_a, Whe0_b = Whe0[:D], Whe0[D:]
    Whe1_a, Whe1_b = Whe1[:D], Whe1[D:]
    Wo_pad = jnp.zeros((D, D), jnp.float32).at[:, :W_out.shape[1]].set(W_out)
    bo_pad = jnp.zeros((1, D), jnp.float32).at[0, :b_out.shape[0]].set(b_out)

    x0, x1, msg = _pre(x_0, x_1, W_hid, b_hid_r, Wn0, bn0_r)
    p1 = _spmm_partials(msg, idxA, zrows)
    msg_he, m1 = _he(x1, p1, Whe0_a, Whe0_b, bhe0_r)
    p0 = _spmm_partials(msg_he, idxB, zrows)
    x0, x1, msg = _upd(x0, p0, gn0_r, gnb0_r, x1, m1, gh0_r, ghb0_r, Wn1, bn1_r)
    p1 = _spmm_partials(msg, idxA, zrows)
    msg_he, m1 = _he(x1, p1, Whe1_a, Whe1_b, bhe1_r)
    p0 = _spmm_partials(msg_he, idxB, zrows)
    out = _fin(x0, p0, gn1_r, gnb1_r, Wo_pad, bo_pad)
    return out[:, :b_out.shape[0]]


# speedup vs baseline: 1.1900x; 1.1900x over previous
import functools

import jax
import jax.numpy as jnp
import numpy as np
from jax import lax
from jax.experimental import pallas as pl
from jax.experimental.pallas import tpu as pltpu
from jax.experimental.pallas import tpu_sc as plsc

N_NODES = 10000
N_EDGES = 10000
NNZ = 320000
D = 128
EPS = 1e-5

_NC, _NS = 2, 16
_W = 128
_NWIN = NNZ // _W
_NWK = _NC * _NS
_WPW = _NWIN // _NWK
_XTRA = _NWIN - _WPW * _NWK
_ACC_ROWS = N_EDGES
_ZCH = _ACC_ROWS // _NS
_CCH = 640
_NBUF = 3

_HIGH = jax.lax.Precision.HIGHEST


def _spmm_partials(msg, idx_win, zrows):
    mesh = plsc.VectorSubcoreMesh(core_axis_name="c", subcore_axis_name="s")

    def body(msg_hbm, idx_hbm, z_hbm, out_hbm, ia0, ia1, ia2, ib0, ib1, ib2,
             r0, r1, r2, g0, g1, g2, sa0, sa1, sa2, sb0, sb1, sb2,
             pa0, pa1, pa2, pb0, pb1, pb2, acc):
        idxa = [ia0, ia1, ia2]
        idxb = [ib0, ib1, ib2]
        rows = [r0, r1, r2]
        gsems = [g0, g1, g2]
        ssems_a = [sa0, sa1, sa2]
        ssems_b = [sb0, sb1, sb2]
        psems_a = [pa0, pa1, pa2]
        psems_b = [pb0, pb1, pb2]
        c = lax.axis_index("c")
        s = lax.axis_index("s")
        wid = s * _NC + c
        base = wid * _WPW
        pltpu.sync_copy(z_hbm, acc.at[pl.ds(s * _ZCH, _ZCH)])
        plsc.subcore_barrier()

        def _scatter_wait(idxs, ssems, b):
            pltpu.make_async_copy(rows[b], acc.at[idxs[b].at[1]],
                                  ssems[b]).wait()

        def _half(j, idxs, psems, ssems, other_idxs, other_ssems, first):
            pf = [pltpu.async_copy(idx_hbm.at[j + b], idxs[b], psems[b])
                  for b in range(_NBUF)]
            g = []
            for b in range(_NBUF):
                if first:
                    @pl.when(j > base)
                    def _():
                        _scatter_wait(other_idxs, other_ssems, b)
                else:
                    _scatter_wait(other_idxs, other_ssems, b)
                pf[b].wait()
                g.append(pltpu.async_copy(msg_hbm.at[idxs[b].at[0]],
                                          rows[b], gsems[b]))
            for b in range(_NBUF):
                g[b].wait()
                pltpu.async_copy(rows[b], acc.at[idxs[b].at[1]],
                                 ssems[b], add=True)

        @pl.loop(0, _WPW, step=2 * _NBUF)
        def _(i):
            _half(base + i, idxa, psems_a, ssems_a, idxb, ssems_b, True)
            _half(base + i + _NBUF, idxb, psems_b, ssems_b, idxa, ssems_a,
                  False)

        for b in range(_NBUF):
            _scatter_wait(idxb, ssems_b, b)

        @pl.when(wid < _XTRA)
        def _():
            j = _NWK * _WPW + wid
            pltpu.sync_copy(idx_hbm.at[j], ia0)
            pltpu.sync_copy(msg_hbm.at[ia0.at[0]], r0)
            pltpu.sync_copy(r0, acc.at[ia0.at[1]], add=True)

        plsc.subcore_barrier()

        @pl.when(s < _NS - 1)
        def _():
            pltpu.sync_copy(acc.at[pl.ds(s * _CCH, _CCH)],
                            out_hbm.at[c].at[pl.ds(s * _CCH, _CCH)])

        @pl.when(s == _NS - 1)
        def _():
            last = (_NS - 1) * _CCH
            pltpu.sync_copy(acc.at[pl.ds(last, N_EDGES - last)],
                            out_hbm.at[c].at[pl.ds(last, N_EDGES - last)])

    k = pl.kernel(
        body,
        out_type=jax.ShapeDtypeStruct((_NC, N_EDGES, D), jnp.float32),
        mesh=mesh,
        scratch_types=(
            [pltpu.VMEM((2, _W), jnp.int32)] * 6
            + [pltpu.VMEM((_W, D), jnp.float32)] * 3
            + [pltpu.SemaphoreType.DMA] * 15
            + [pltpu.VMEM_SHARED((_ACC_ROWS, D), jnp.float32)]
        ),
    )
    return k(msg, idx_win, zrows)


_RB = 1000
_GRID = N_NODES // _RB


def _pre_body(x0_ref, x1_ref, wh_ref, bh_ref, wn_ref, bn_ref,
              h0_ref, h1_ref, msg_ref):
    h0 = jnp.dot(x0_ref[...], wh_ref[...], precision=_HIGH) + bh_ref[...]
    h1 = jnp.dot(x1_ref[...], wh_ref[...], precision=_HIGH) + bh_ref[...]
    h0_ref[...] = h0
    h1_ref[...] = h1
    msg_ref[...] = jax.nn.sigmoid(
        jnp.dot(h0, wn_ref[...], precision=_HIGH) + bn_ref[...])


def _pre(x_0, x_1, W_hid, b_hid, Wn, bn):
    row = lambda i: (i, 0)
    fixed = lambda i: (0, 0)
    return pl.pallas_call(
        _pre_body,
        grid=(_GRID,),
        in_specs=[
            pl.BlockSpec((_RB, D), row), pl.BlockSpec((_RB, D), row),
            pl.BlockSpec((D, D), fixed), pl.BlockSpec((1, D), fixed),
            pl.BlockSpec((D, D), fixed), pl.BlockSpec((1, D), fixed),
        ],
        out_specs=[pl.BlockSpec((_RB, D), row)] * 3,
        out_shape=[jax.ShapeDtypeStruct((N_NODES, D), jnp.float32)] * 3,
    )(x_0, x_1, W_hid, b_hid, Wn, bn)


def _he_body(x1_ref, p_ref, wa_ref, wb_ref, bhe_ref, msg_ref, m1_ref):
    m1 = p_ref[0] + p_ref[1]
    m1_ref[...] = m1
    msg_ref[...] = jax.nn.sigmoid(
        jnp.dot(x1_ref[...], wa_ref[...], precision=_HIGH)
        + jnp.dot(m1, wb_ref[...], precision=_HIGH) + bhe_ref[...])


def _he(x1, p, Whe_a, Whe_b, bhe):
    row = lambda i: (i, 0)
    fixed = lambda i: (0, 0)
    return pl.pallas_call(
        _he_body,
        grid=(_GRID,),
        in_specs=[
            pl.BlockSpec((_RB, D), row),
            pl.BlockSpec((_NC, _RB, D), lambda i: (0, i, 0)),
            pl.BlockSpec((D, D), fixed), pl.BlockSpec((D, D), fixed),
            pl.BlockSpec((1, D), fixed),
        ],
        out_specs=[pl.BlockSpec((_RB, D), row)] * 2,
        out_shape=[jax.ShapeDtypeStruct((N_EDGES, D), jnp.float32)] * 2,
    )(x1, p, Whe_a, Whe_b, bhe)


_BN_SCALE = float(1.0 / np.sqrt(1.0 + EPS))


def _upd_body(x0_ref, p_ref, gn_ref, gnb_ref, x1_ref, m1_ref, gh_ref, ghb_ref,
              wn_ref, bn_ref, x0o_ref, x1o_ref, msg_ref):
    x0n = jax.nn.sigmoid(x0_ref[...] * (gn_ref[...] * _BN_SCALE) + gnb_ref[...]
                         + p_ref[0] + p_ref[1])
    x1n = jax.nn.sigmoid(x1_ref[...] * (gh_ref[...] * _BN_SCALE) + ghb_ref[...]
                         + m1_ref[...])
    x0o_ref[...] = x0n
    x1o_ref[...] = x1n
    msg_ref[...] = jax.nn.sigmoid(
        jnp.dot(x0n, wn_ref[...], precision=_HIGH) + bn_ref[...])


def _upd(x0, p, gn, gnb, x1, m1, gh, ghb, Wn, bn):
    row = lambda i: (i, 0)
    fixed = lambda i: (0, 0)
    return pl.pallas_call(
        _upd_body,
        grid=(_GRID,),
        in_specs=[
            pl.BlockSpec((_RB, D), row),
            pl.BlockSpec((_NC, _RB, D), lambda i: (0, i, 0)),
            pl.BlockSpec((1, D), fixed), pl.BlockSpec((1, D), fixed),
            pl.BlockSpec((_RB, D), row), pl.BlockSpec((_RB, D), row),
            pl.BlockSpec((1, D), fixed), pl.BlockSpec((1, D), fixed),
            pl.BlockSpec((D, D), fixed), pl.BlockSpec((1, D), fixed),
        ],
        out_specs=[pl.BlockSpec((_RB, D), row)] * 3,
        out_shape=[jax.ShapeDtypeStruct((N_NODES, D), jnp.float32)] * 3,
    )(x0, p, gn, gnb, x1, m1, gh, ghb, Wn, bn)


def _fin_body(x0_ref, p_ref, gn_ref, gnb_ref, wo_ref, bo_ref, out_ref):
    x0n = jax.nn.sigmoid(x0_ref[...] * (gn_ref[...] * _BN_SCALE) + gnb_ref[...]
                         + p_ref[0] + p_ref[1])
    out_ref[...] = jnp.dot(x0n, wo_ref[...], precision=_HIGH) + bo_ref[...]


def _fin(x0, p, gn, gnb, Wo_pad, bo_pad):
    row = lambda i: (i, 0)
    fixed = lambda i: (0, 0)
    return pl.pallas_call(
        _fin_body,
        grid=(_GRID,),
        in_specs=[
            pl.BlockSpec((_RB, D), row),
            pl.BlockSpec((_NC, _RB, D), lambda i: (0, i, 0)),
            pl.BlockSpec((1, D), fixed), pl.BlockSpec((1, D), fixed),
            pl.BlockSpec((D, D), fixed), pl.BlockSpec((1, D), fixed),
        ],
        out_specs=pl.BlockSpec((_RB, D), row),
        out_shape=jax.ShapeDtypeStruct((N_NODES, D), jnp.float32),
    )(x0, p, gn, gnb, Wo_pad, bo_pad)


def kernel(x_0, x_1, W_hid, b_hid, Wn0, bn0, Whe0, bhe0, gamma_n0, beta_n0,
           gamma_h0, beta_h0, Wn1, bn1, Whe1, bhe1, gamma_n1, beta_n1,
           gamma_h1, beta_h1, W_out, b_out, incidence_1):
    node_idx = incidence_1[0]
    he_idx = incidence_1[1]

    def _windows(src, dst):
        return jnp.stack([src.reshape(_NWIN, _W), dst.reshape(_NWIN, _W)],
                         axis=1)

    idxA = _windows(node_idx, he_idx)
    idxB = _windows(he_idx, node_idx)
    zrows = jnp.zeros((_ZCH, D), jnp.float32)

    b_hid_r = b_hid.reshape(1, D)
    bn0_r = bn0.reshape(1, D)
    bn1_r = bn1.reshape(1, D)
    bhe0_r = bhe0.reshape(1, D)
    bhe1_r = bhe1.reshape(1, D)
    gn0_r = gamma_n0.reshape(1, D)
    gnb0_r = beta_n0.reshape(1, D)
    gh0_r = gamma_h0.reshape(1, D)
    ghb0_r = beta_h0.reshape(1, D)
    gn1_r = gamma_n1.reshape(1, D)
    gnb1_r = beta_n1.reshape(1, D)
    gh1_r = gamma_h1.reshape(1, D)
    ghb1_r = beta_h1.reshape(1, D)
    Whe0_a, Whe0_b = Whe0[:D], Whe0[D:]
    Whe1_a, Whe1_b = Whe1[:D], Whe1[D:]
    Wo_pad = jnp.zeros((D, D), jnp.float32).at[:, :W_out.shape[1]].set(W_out)
    bo_pad = jnp.zeros((1, D), jnp.float32).at[0, :b_out.shape[0]].set(b_out)

    x0, x1, msg = _pre(x_0, x_1, W_hid, b_hid_r, Wn0, bn0_r)
    p1 = _spmm_partials(msg, idxA, zrows)
    msg_he, m1 = _he(x1, p1, Whe0_a, Whe0_b, bhe0_r)
    p0 = _spmm_partials(msg_he, idxB, zrows)
    x0, x1, msg = _upd(x0, p0, gn0_r, gnb0_r, x1, m1, gh0_r, ghb0_r, Wn1, bn1_r)
    p1 = _spmm_partials(msg, idxA, zrows)
    msg_he, m1 = _he(x1, p1, Whe1_a, Whe1_b, bhe1_r)
    p0 = _spmm_partials(msg_he, idxB, zrows)
    out = _fin(x0, p0, gn1_r, gnb1_r, Wo_pad, bo_pad)
    return out[:, :b_out.shape[0]]


# speedup vs baseline: 1.2722x; 1.0691x over previous
import functools

import jax
import jax.numpy as jnp
import numpy as np
from jax import lax
from jax.experimental import pallas as pl
from jax.experimental.pallas import tpu as pltpu
from jax.experimental.pallas import tpu_sc as plsc

N_NODES = 10000
N_EDGES = 10000
NNZ = 320000
D = 128
EPS = 1e-5

_NC, _NS = 2, 16
_W = 128
_NWIN = NNZ // _W
_NWK = _NC * _NS
_WPW = _NWIN // _NWK
_XTRA = _NWIN - _WPW * _NWK
_ACC_ROWS = N_EDGES
_ZCH = _ACC_ROWS // _NS
_CCH = 640
_NBUF = 3

_HIGH = jax.lax.Precision.HIGHEST


def _spmm_partials(msg, idx_win, zrows):
    mesh = plsc.VectorSubcoreMesh(core_axis_name="c", subcore_axis_name="s")

    def body(msg_hbm, idx_hbm, z_hbm, out_hbm, i0, i1, i2, r0, r1, r2,
             g0, g1, g2, s0, s1, s2, acc):
        idxs = [i0, i1, i2]
        rows = [r0, r1, r2]
        gsems = [g0, g1, g2]
        ssems = [s0, s1, s2]
        c = lax.axis_index("c")
        s = lax.axis_index("s")
        wid = s * _NC + c
        base = wid * _WPW
        pltpu.sync_copy(z_hbm, acc.at[pl.ds(s * _ZCH, _ZCH)])
        plsc.subcore_barrier()

        def _scatter_wait(b):
            pltpu.make_async_copy(rows[b], acc.at[idxs[b].at[1]],
                                  ssems[b]).wait()

        @pl.loop(0, _WPW, step=_NBUF)
        def _(i):
            g = []
            for b in range(_NBUF):
                @pl.when(i > 0)
                def _():
                    _scatter_wait(b)
                pltpu.sync_copy(idx_hbm.at[base + i + b], idxs[b])
                g.append(pltpu.async_copy(msg_hbm.at[idxs[b].at[0]],
                                          rows[b], gsems[b]))
            for b in range(_NBUF):
                g[b].wait()
                pltpu.async_copy(rows[b], acc.at[idxs[b].at[1]],
                                 ssems[b], add=True)

        for b in range(_NBUF):
            _scatter_wait(b)

        @pl.when(wid < _XTRA)
        def _():
            j = _NWK * _WPW + wid
            pltpu.sync_copy(idx_hbm.at[j], i0)
            pltpu.sync_copy(msg_hbm.at[i0.at[0]], r0)
            pltpu.sync_copy(r0, acc.at[i0.at[1]], add=True)

        plsc.subcore_barrier()

        @pl.when(s < _NS - 1)
        def _():
            pltpu.sync_copy(acc.at[pl.ds(s * _CCH, _CCH)],
                            out_hbm.at[c].at[pl.ds(s * _CCH, _CCH)])

        @pl.when(s == _NS - 1)
        def _():
            last = (_NS - 1) * _CCH
            pltpu.sync_copy(acc.at[pl.ds(last, N_EDGES - last)],
                            out_hbm.at[c].at[pl.ds(last, N_EDGES - last)])

    k = pl.kernel(
        body,
        out_type=jax.ShapeDtypeStruct((_NC, N_EDGES, D), jnp.float32),
        mesh=mesh,
        scratch_types=[
            pltpu.VMEM((2, _W), jnp.int32),
            pltpu.VMEM((2, _W), jnp.int32),
            pltpu.VMEM((2, _W), jnp.int32),
            pltpu.VMEM((_W, D), jnp.float32),
            pltpu.VMEM((_W, D), jnp.float32),
            pltpu.VMEM((_W, D), jnp.float32),
            pltpu.SemaphoreType.DMA,
            pltpu.SemaphoreType.DMA,
            pltpu.SemaphoreType.DMA,
            pltpu.SemaphoreType.DMA,
            pltpu.SemaphoreType.DMA,
            pltpu.SemaphoreType.DMA,
            pltpu.VMEM_SHARED((_ACC_ROWS, D), jnp.float32),
        ],
    )
    return k(msg, idx_win, zrows)


_RB = 1000
_GRID = N_NODES // _RB


def _pre_body(x0_ref, x1_ref, wh_ref, bh_ref, wn_ref, bn_ref,
              h0_ref, h1_ref, msg_ref):
    h0 = jnp.dot(x0_ref[...], wh_ref[...], precision=_HIGH) + bh_ref[...]
    h1 = jnp.dot(x1_ref[...], wh_ref[...], precision=_HIGH) + bh_ref[...]
    h0_ref[...] = h0
    h1_ref[...] = h1
    msg_ref[...] = jax.nn.sigmoid(
        jnp.dot(h0, wn_ref[...], precision=_HIGH) + bn_ref[...])


def _pre(x_0, x_1, W_hid, b_hid, Wn, bn):
    row = lambda i: (i, 0)
    fixed = lambda i: (0, 0)
    return pl.pallas_call(
        _pre_body,
        grid=(_GRID,),
        in_specs=[
            pl.BlockSpec((_RB, D), row), pl.BlockSpec((_RB, D), row),
            pl.BlockSpec((D, D), fixed), pl.BlockSpec((1, D), fixed),
            pl.BlockSpec((D, D), fixed), pl.BlockSpec((1, D), fixed),
        ],
        out_specs=[pl.BlockSpec((_RB, D), row)] * 3,
        out_shape=[jax.ShapeDtypeStruct((N_NODES, D), jnp.float32)] * 3,
    )(x_0, x_1, W_hid, b_hid, Wn, bn)


def _he_body(x1_ref, p_ref, wa_ref, wb_ref, bhe_ref, msg_ref, m1_ref):
    m1 = p_ref[0] + p_ref[1]
    m1_ref[...] = m1
    msg_ref[...] = jax.nn.sigmoid(
        jnp.dot(x1_ref[...], wa_ref[...], precision=_HIGH)
        + jnp.dot(m1, wb_ref[...], precision=_HIGH) + bhe_ref[...])


def _he(x1, p, Whe_a, Whe_b, bhe):
    row = lambda i: (i, 0)
    fixed = lambda i: (0, 0)
    return pl.pallas_call(
        _he_body,
        grid=(_GRID,),
        in_specs=[
            pl.BlockSpec((_RB, D), row),
            pl.BlockSpec((_NC, _RB, D), lambda i: (0, i, 0)),
            pl.BlockSpec((D, D), fixed), pl.BlockSpec((D, D), fixed),
            pl.BlockSpec((1, D), fixed),
        ],
        out_specs=[pl.BlockSpec((_RB, D), row)] * 2,
        out_shape=[jax.ShapeDtypeStruct((N_EDGES, D), jnp.float32)] * 2,
    )(x1, p, Whe_a, Whe_b, bhe)


_BN_SCALE = float(1.0 / np.sqrt(1.0 + EPS))


def _upd_body(x0_ref, p_ref, gn_ref, gnb_ref, x1_ref, m1_ref, gh_ref, ghb_ref,
              wn_ref, bn_ref, x0o_ref, x1o_ref, msg_ref):
    x0n = jax.nn.sigmoid(x0_ref[...] * (gn_ref[...] * _BN_SCALE) + gnb_ref[...]
                         + p_ref[0] + p_ref[1])
    x1n = jax.nn.sigmoid(x1_ref[...] * (gh_ref[...] * _BN_SCALE) + ghb_ref[...]
                         + m1_ref[...])
    x0o_ref[...] = x0n
    x1o_ref[...] = x1n
    msg_ref[...] = jax.nn.sigmoid(
        jnp.dot(x0n, wn_ref[...], precision=_HIGH) + bn_ref[...])


def _upd(x0, p, gn, gnb, x1, m1, gh, ghb, Wn, bn):
    row = lambda i: (i, 0)
    fixed = lambda i: (0, 0)
    return pl.pallas_call(
        _upd_body,
        grid=(_GRID,),
        in_specs=[
            pl.BlockSpec((_RB, D), row),
            pl.BlockSpec((_NC, _RB, D), lambda i: (0, i, 0)),
            pl.BlockSpec((1, D), fixed), pl.BlockSpec((1, D), fixed),
            pl.BlockSpec((_RB, D), row), pl.BlockSpec((_RB, D), row),
            pl.BlockSpec((1, D), fixed), pl.BlockSpec((1, D), fixed),
            pl.BlockSpec((D, D), fixed), pl.BlockSpec((1, D), fixed),
        ],
        out_specs=[pl.BlockSpec((_RB, D), row)] * 3,
        out_shape=[jax.ShapeDtypeStruct((N_NODES, D), jnp.float32)] * 3,
    )(x0, p, gn, gnb, x1, m1, gh, ghb, Wn, bn)


def _fin_body(x0_ref, p_ref, gn_ref, gnb_ref, wo_ref, bo_ref, out_ref):
    x0n = jax.nn.sigmoid(x0_ref[...] * (gn_ref[...] * _BN_SCALE) + gnb_ref[...]
                         + p_ref[0] + p_ref[1])
    out_ref[...] = jnp.dot(x0n, wo_ref[...], precision=_HIGH) + bo_ref[...]


def _fin(x0, p, gn, gnb, Wo_pad, bo_pad):
    row = lambda i: (i, 0)
    fixed = lambda i: (0, 0)
    return pl.pallas_call(
        _fin_body,
        grid=(_GRID,),
        in_specs=[
            pl.BlockSpec((_RB, D), row),
            pl.BlockSpec((_NC, _RB, D), lambda i: (0, i, 0)),
            pl.BlockSpec((1, D), fixed), pl.BlockSpec((1, D), fixed),
            pl.BlockSpec((D, D), fixed), pl.BlockSpec((1, D), fixed),
        ],
        out_specs=pl.BlockSpec((_RB, D), row),
        out_shape=jax.ShapeDtypeStruct((N_NODES, D), jnp.float32),
    )(x0, p, gn, gnb, Wo_pad, bo_pad)


def kernel(x_0, x_1, W_hid, b_hid, Wn0, bn0, Whe0, bhe0, gamma_n0, beta_n0,
           gamma_h0, beta_h0, Wn1, bn1, Whe1, bhe1, gamma_n1, beta_n1,
           gamma_h1, beta_h1, W_out, b_out, incidence_1):
    node_idx = incidence_1[0]
    he_idx = incidence_1[1]

    def _windows(src, dst):
        return jnp.stack([src.reshape(_NWIN, _W), dst.reshape(_NWIN, _W)],
                         axis=1)

    idxA = _windows(node_idx, he_idx)
    idxB = _windows(he_idx, node_idx)
    zrows = jnp.zeros((_ZCH, D), jnp.float32)

    b_hid_r = b_hid.reshape(1, D)
    bn0_r = bn0.reshape(1, D)
    bn1_r = bn1.reshape(1, D)
    bhe0_r = bhe0.reshape(1, D)
    bhe1_r = bhe1.reshape(1, D)
    gn0_r = gamma_n0.reshape(1, D)
    gnb0_r = beta_n0.reshape(1, D)
    gh0_r = gamma_h0.reshape(1, D)
    ghb0_r = beta_h0.reshape(1, D)
    gn1_r = gamma_n1.reshape(1, D)
    gnb1_r = beta_n1.reshape(1, D)
    gh1_r = gamma_h1.reshape(1, D)
    ghb1_r = beta_h1.reshape(1, D)
    Whe0_a, Whe0_b = Whe0[:D], Whe0[D:]
    Whe1_a, Whe1_b = Whe1[:D], Whe1[D:]
    Wo_pad = jnp.zeros((D, D), jnp.float32).at[:, :W_out.shape[1]].set(W_out)
    bo_pad = jnp.zeros((1, D), jnp.float32).at[0, :b_out.shape[0]].set(b_out)

    x0, x1, msg = _pre(x_0, x_1, W_hid, b_hid_r, Wn0, bn0_r)
    p1 = _spmm_partials(msg, idxA, zrows)
    msg_he, m1 = _he(x1, p1, Whe0_a, Whe0_b, bhe0_r)
    p0 = _spmm_partials(msg_he, idxB, zrows)
    x0, x1, msg = _upd(x0, p0, gn0_r, gnb0_r, x1, m1, gh0_r, ghb0_r, Wn1, bn1_r)
    p1 = _spmm_partials(msg, idxA, zrows)
    msg_he, m1 = _he(x1, p1, Whe1_a, Whe1_b, bhe1_r)
    p0 = _spmm_partials(msg_he, idxB, zrows)
    out = _fin(x0, p0, gn1_r, gnb1_r, Wo_pad, bo_pad)
    return out[:, :b_out.shape[0]]
